# baseline (speedup 1.0000x reference)
import functools

import jax
import jax.numpy as jnp
from jax import lax
from jax.experimental import pallas as pl
from jax.experimental.pallas import tpu as pltpu
from jax.experimental.pallas import tpu_sc as plsc

VOCAB = 100000
EMBED = 64
B = 4096
L = 200

_info = plsc.get_sparse_core_info()
NC, NS = _info.num_cores, _info.num_subcores
NW = NC * NS
BW = B // NW
LB = 2
NBUF = 2
NL = L // LB

_mesh = plsc.VectorSubcoreMesh(core_axis_name="c", subcore_axis_name="s")


@functools.partial(
    pl.kernel,
    mesh=_mesh,
    out_type=jax.ShapeDtypeStruct((L * EMBED // 8, NW, 8 * BW), jnp.float32),
    scratch_types=[
        pltpu.VMEM((L // 8, 8, BW), jnp.int32),
        pltpu.VMEM((NBUF, LB, BW, EMBED), jnp.float32),
        pltpu.VMEM((NBUF, LB * EMBED // 8, 8 * BW), jnp.float32),
        pltpu.SemaphoreType.DMA,
        pltpu.SemaphoreType.DMA,
        pltpu.SemaphoreType.DMA,
        pltpu.SemaphoreType.DMA,
    ],
    compiler_params=pltpu.CompilerParams(
        use_tc_tiling_on_sc=False, needs_layout_passes=False
    ),
)
def _gather_kernel(idx_hbm, table_hbm, out_hbm, idx_v, g_v, t_v, gs0, gs1, ws0, ws1):
    gsem = (gs0, gs1)
    wsem = (ws0, ws1)
    wid = lax.axis_index("s") * NC + lax.axis_index("c")
    pltpu.sync_copy(idx_hbm.at[:, wid], idx_v)

    lane = lax.iota(jnp.int32, 16)
    row_vecs = [
        [
            lax.shift_right_logical(lane, 3) + (ll * (EMBED // 8) + 2 * k)
            for k in range(EMBED // 16)
        ]
        for ll in range(LB)
    ]
    jlo_vec = lax.shift_left(lax.bitwise_and(lane, 7), 7)

    def idx_list(l, ll):
        return idx_v.at[(l + ll) // 8, (l + ll) % 8]

    def fire_gather(l, ll, b):
        pltpu.async_copy(table_hbm.at[idx_list(l, ll)], g_v.at[b, ll], gsem[b])

    def wait_gather(l, ll, b):
        pltpu.make_async_copy(
            table_hbm.at[idx_list(l, ll)], g_v.at[b, ll], gsem[b]
        ).wait()

    def out_slice(l):
        nrow = LB * EMBED // 8
        return out_hbm.at[pl.ds(pl.multiple_of(l * (EMBED // 8), nrow), nrow), wid]

    def fire_write(l, b):
        pltpu.async_copy(t_v.at[b], out_slice(l), wsem[b])

    def wait_write(l, b):
        pltpu.make_async_copy(t_v.at[b], out_slice(l), wsem[b]).wait()

    UN = 4

    def transpose_block(b):
        dst = t_v.at[b]

        def body(c0, carry):
            for u in range(UN):
                c = c0 * UN + u
                inner = jlo_vec + lax.broadcast(c, (16,))
                for ll in range(LB):
                    for k in range(EMBED // 16):
                        vec = g_v[b, ll, c, pl.ds(k * 16, 16)]
                        plsc.store_scatter(dst, [row_vecs[ll][k], inner], vec)
            return carry

        lax.fori_loop(0, BW // UN, body, 0)

    for b in range(NBUF):
        for ll in range(LB):
            fire_gather(b * LB, ll, b)

    def group(g, carry):
        for b in range(NBUF):
            l = (g * NBUF + b) * LB
            for ll in range(LB):
                wait_gather(l, ll, b)
            transpose_block(b)
            fire_write(l, b)
            wait_write(l, b)
            for ll in range(LB):
                fire_gather(l + NBUF * LB, ll, b)
        return carry

    lax.fori_loop(0, NL // NBUF - 1, group, 0)

    for b in range(NBUF):
        l = (NL // NBUF - 1) * NBUF * LB + b * LB
        for ll in range(LB):
            wait_gather(l, ll, b)
        transpose_block(b)
        fire_write(l, b)
    for b in range(NBUF):
        l = (NL // NBUF - 1) * NBUF * LB + b * LB
        wait_write(l, b)


def kernel(indices, table):
    idx4 = (
        indices.astype(jnp.int32)
        .T.reshape(L // 8, 8, NW, BW)
        .transpose(0, 2, 1, 3)
    )
    out = _gather_kernel(idx4, table)
    return (
        out.reshape(L, EMBED // 8, NW, 8, BW)
        .transpose(2, 4, 0, 1, 3)
        .reshape(B, L, EMBED)
    )


# speedup vs baseline: 2.0710x; 2.0710x over previous
import functools

import jax
import jax.numpy as jnp
from jax import lax
from jax.experimental import pallas as pl
from jax.experimental.pallas import tpu as pltpu
from jax.experimental.pallas import tpu_sc as plsc

VOCAB = 100000
EMBED = 64
B = 4096
L = 200

_info = plsc.get_sparse_core_info()
NC, NS = _info.num_cores, _info.num_subcores
NW = NC * NS
BW = B // NW
LB = 2
NBUF = 2
NL = L // LB

_mesh = plsc.VectorSubcoreMesh(core_axis_name="c", subcore_axis_name="s")


@functools.partial(
    pl.kernel,
    mesh=_mesh,
    out_type=jax.ShapeDtypeStruct((L * EMBED // 8, NW, 8 * BW), jnp.float32),
    scratch_types=[
        pltpu.VMEM((L // 8, 8, BW), jnp.int32),
        pltpu.VMEM((NBUF, LB, BW, EMBED), jnp.float32),
        pltpu.VMEM((NBUF, LB * EMBED // 8, 8 * BW), jnp.float32),
        pltpu.SemaphoreType.DMA,
        pltpu.SemaphoreType.DMA,
        pltpu.SemaphoreType.DMA,
        pltpu.SemaphoreType.DMA,
    ],
    compiler_params=pltpu.CompilerParams(
        use_tc_tiling_on_sc=False, needs_layout_passes=False
    ),
)
def _gather_kernel(idx_hbm, table_hbm, out_hbm, idx_v, g_v, t_v, gs0, gs1, ws0, ws1):
    gsem = (gs0, gs1)
    wsem = (ws0, ws1)
    wid = lax.axis_index("s") * NC + lax.axis_index("c")
    pltpu.sync_copy(idx_hbm.at[:, wid], idx_v)

    lane = lax.iota(jnp.int32, 16)
    perms = [lax.bitwise_and(lane + d, 15) for d in range(16)]
    perm_hi = [lax.shift_right_logical(p, 3) for p in perms]
    perm_lo128 = [lax.shift_left(lax.bitwise_and(p, 7), 7) for p in perms]

    def idx_list(l, ll):
        return idx_v.at[(l + ll) // 8, (l + ll) % 8]

    def fire_gather(l, ll, b):
        pltpu.async_copy(table_hbm.at[idx_list(l, ll)], g_v.at[b, ll], gsem[b])

    def wait_gather(l, ll, b):
        pltpu.make_async_copy(
            table_hbm.at[idx_list(l, ll)], g_v.at[b, ll], gsem[b]
        ).wait()

    def out_slice(l):
        nrow = LB * EMBED // 8
        return out_hbm.at[pl.ds(pl.multiple_of(l * (EMBED // 8), nrow), nrow), wid]

    def fire_write(l, b):
        pltpu.async_copy(t_v.at[b], out_slice(l), wsem[b])

    def wait_write(l, b):
        pltpu.make_async_copy(t_v.at[b], out_slice(l), wsem[b]).wait()

    def transpose_block(b):
        dst = t_v.at[b]

        def body(c0, carry):
            c_vec = lane + lax.broadcast(c0 * 16, (16,))
            for d in range(16):
                inner = perm_lo128[d] + c_vec
                for ll in range(LB):
                    src = g_v.at[b, ll]
                    for k in range(EMBED // 16):
                        j_vec = perms[d] + (k * 16)
                        row = perm_hi[d] + (ll * (EMBED // 8) + 2 * k)
                        vec = plsc.load_gather(src, [c_vec, j_vec])
                        plsc.store_scatter(dst, [row, inner], vec)
            return carry

        lax.fori_loop(0, BW // 16, body, 0)

    for b in range(NBUF):
        for ll in range(LB):
            fire_gather(b * LB, ll, b)

    def group(g, carry):
        for b in range(NBUF):
            l = (g * NBUF + b) * LB
            for ll in range(LB):
                wait_gather(l, ll, b)
            transpose_block(b)
            fire_write(l, b)
            wait_write(l, b)
            for ll in range(LB):
                fire_gather(l + NBUF * LB, ll, b)
        return carry

    lax.fori_loop(0, NL // NBUF - 1, group, 0)

    for b in range(NBUF):
        l = (NL // NBUF - 1) * NBUF * LB + b * LB
        for ll in range(LB):
            wait_gather(l, ll, b)
        transpose_block(b)
        fire_write(l, b)
    for b in range(NBUF):
        l = (NL // NBUF - 1) * NBUF * LB + b * LB
        wait_write(l, b)


def kernel(indices, table):
    idx4 = (
        indices.astype(jnp.int32)
        .T.reshape(L // 8, 8, NW, BW)
        .transpose(0, 2, 1, 3)
    )
    out = _gather_kernel(idx4, table)
    return (
        out.reshape(L, EMBED // 8, NW, 8, BW)
        .transpose(2, 4, 0, 1, 3)
        .reshape(B, L, EMBED)
    )
